# baseline (speedup 1.0000x reference)
import jax
import jax.numpy as jnp
from jax.experimental import pallas as pl
from jax.experimental.pallas import tpu as pltpu

_KB = 4
_MODES = 32
_W = 64


def _subnet(xin, f_mat, if_aug, w0k, layer_ws, w1k, b1k, w2k, b2k):
    f32 = jnp.float32
    bf16 = jnp.bfloat16
    m = _MODES
    w = _W
    kb = _KB

    hb = jnp.dot(xin, w0k, preferred_element_type=f32).astype(bf16)

    n_layers = len(layer_ws)
    for l, (v2, wp_lo, wp_hi) in enumerate(layer_ws):
        xhat = jnp.dot(f_mat, hb, preferred_element_type=f32)
        xb = xhat.astype(bf16)
        a3 = xb.reshape(2 * m, kb, w)
        x3d = jnp.concatenate([a3[:m], a3[m:]], axis=2)
        out3 = jnp.einsum("mec,mco->meo", x3d, v2,
                          preferred_element_type=f32)
        top = out3[:, :, :w].reshape(m, kb * w)
        bot = out3[:, :, w:].reshape(m, kb * w)
        out_stack = jnp.concatenate([top, bot], axis=0).astype(bf16)
        lhs = jnp.concatenate([hb, if_aug], axis=1)
        rhs = jnp.concatenate([wp_lo, out_stack, wp_hi], axis=0)
        h = jnp.dot(lhs, rhs, preferred_element_type=f32)
        if l != n_layers - 1:
            h = jnp.maximum(h, 0.0)
        hb = h.astype(bf16)

    h1 = jnp.dot(hb, w1k, preferred_element_type=f32) + b1k
    h1 = jnp.maximum(h1, 0.0).astype(bf16)
    return jnp.dot(h1, w2k, preferred_element_type=f32) + b2k


def _body(xa_ref, xb_ref, f_ref, ifa_ref, w0_ref,
          v2a_ref, wpa_lo_ref, wpa_hi_ref,
          v2b_ref, wpb_lo_ref, wpb_hi_ref,
          v2c_ref, wpc_lo_ref, wpc_hi_ref,
          w1_ref, b1_ref, w2_ref, b2_ref, o_ref):
    f_mat = f_ref[...]
    if_aug = ifa_ref[...]
    w0k = w0_ref[...]
    layer_ws = ((v2a_ref[...], wpa_lo_ref[...], wpa_hi_ref[...]),
                (v2b_ref[...], wpb_lo_ref[...], wpb_hi_ref[...]),
                (v2c_ref[...], wpc_lo_ref[...], wpc_hi_ref[...]))
    head = (w1_ref[...], b1_ref[...], w2_ref[...], b2_ref[...])
    ya = _subnet(xa_ref[...], f_mat, if_aug, w0k, layer_ws, *head)
    yb = _subnet(xb_ref[...], f_mat, if_aug, w0k, layer_ws, *head)
    o_ref[...] = jnp.concatenate([ya, yb], axis=1)


def _blockdiag(wmat, k):
    return jnp.kron(jnp.eye(k, dtype=wmat.dtype), wmat)


@jax.jit
def kernel(x, dft_fwd, dft_inv, w0, b0, w1, b1, w2, b2,
           v2_0, wp_0, bp_0, v2_1, wp_1, bp_1, v2_2, wp_2, bp_2):
    bf16 = jnp.bfloat16
    f32 = jnp.float32
    B, s, cin0 = x.shape
    kb = _KB
    G = B // kb
    G2 = G // 2

    x4 = (x.reshape(G, kb, s, cin0).transpose(0, 2, 1, 3)
          .reshape(G, s, kb * cin0).astype(bf16))
    x4 = jnp.concatenate([x4, jnp.ones((G, s, 1), bf16)], axis=2)
    xa, xb = x4[:G2], x4[G2:]

    f_mat = dft_fwd.astype(bf16)
    if_aug = jnp.concatenate(
        [dft_inv, jnp.ones((s, 1), f32), jnp.zeros((s, 63), f32)],
        axis=1).astype(bf16)
    w0k = jnp.concatenate([_blockdiag(w0, kb), jnp.tile(b0, (1, kb))],
                          axis=0).astype(bf16)
    w1k = _blockdiag(w1, kb).astype(bf16)
    b1k = jnp.tile(b1, (1, kb))
    w2k = _blockdiag(w2, kb).astype(bf16)
    b2k = jnp.tile(b2, (1, kb))

    inputs = [xa, xb, f_mat, if_aug, w0k]
    for v2, wp, bp in ((v2_0, wp_0, bp_0), (v2_1, wp_1, bp_1),
                       (v2_2, wp_2, bp_2)):
        wp_lo = _blockdiag(wp, kb)
        wp_hi = jnp.concatenate(
            [jnp.tile(bp, (1, kb)), jnp.zeros((63, kb * w0.shape[1]), f32)],
            axis=0)
        inputs += [v2.astype(bf16), wp_lo.astype(bf16), wp_hi.astype(bf16)]
    inputs += [w1k, b1k, w2k, b2k]

    def full(arr):
        shp = tuple(arr.shape)
        return pl.BlockSpec(shp, lambda b, _r=len(shp): (0,) * _r)

    xspec = pl.BlockSpec((pl.Squeezed(), s, kb * cin0 + 1), lambda b: (b, 0, 0))
    in_specs = [xspec, xspec] + [full(a) for a in inputs[2:]]

    out = pl.pallas_call(
        _body,
        out_shape=jax.ShapeDtypeStruct((G2, s, 2 * kb), f32),
        grid=(G2,),
        in_specs=in_specs,
        out_specs=pl.BlockSpec((pl.Squeezed(), s, 2 * kb), lambda b: (b, 0, 0)),
        compiler_params=pltpu.CompilerParams(
            dimension_semantics=("parallel",),
            vmem_limit_bytes=48 * 1024 * 1024,
        ),
    )(*inputs)

    ya = out[:, :, :kb].transpose(0, 2, 1).reshape(B // 2, s, 1)
    yb = out[:, :, kb:].transpose(0, 2, 1).reshape(B // 2, s, 1)
    return jnp.concatenate([ya, yb], axis=0)


# speedup vs baseline: 1.0725x; 1.0725x over previous
import jax
import jax.numpy as jnp
from jax.experimental import pallas as pl
from jax.experimental.pallas import tpu as pltpu

_KB = 4
_MODES = 32
_W = 64


def _body(x_ref, f_ref, ifa_ref, w0_ref,
          v2a_ref, wpa_lo_ref, wpa_hi_ref,
          v2b_ref, wpb_lo_ref, wpb_hi_ref,
          v2c_ref, wpc_lo_ref, wpc_hi_ref,
          w1_ref, b1_ref, w2_ref, b2_ref, o_ref):
    f32 = jnp.float32
    bf16 = jnp.bfloat16
    m = _MODES
    w = _W
    kb = _KB

    hb = jnp.dot(x_ref[...], w0_ref[...], preferred_element_type=f32).astype(bf16)

    if_aug = ifa_ref[...]
    layers = ((v2a_ref, wpa_lo_ref, wpa_hi_ref, False),
              (v2b_ref, wpb_lo_ref, wpb_hi_ref, False),
              (v2c_ref, wpc_lo_ref, wpc_hi_ref, True))
    for v2_ref, wp_lo_ref, wp_hi_ref, last in layers:
        xhat = jnp.dot(f_ref[...], hb, preferred_element_type=f32)
        xb = xhat.astype(bf16)
        a3 = xb.reshape(2 * m, kb, w)
        x3d = jnp.concatenate([a3[:m], a3[m:]], axis=2)
        out3 = jnp.einsum("mec,mco->meo", x3d, v2_ref[...],
                          preferred_element_type=f32)
        top = out3[:, :, :w].reshape(m, kb * w)
        bot = out3[:, :, w:].reshape(m, kb * w)
        out_stack = jnp.concatenate([top, bot], axis=0).astype(bf16)
        lhs = jnp.concatenate([hb, if_aug], axis=1)
        rhs = jnp.concatenate([wp_lo_ref[...], out_stack, wp_hi_ref[...]],
                              axis=0)
        h = jnp.dot(lhs, rhs, preferred_element_type=f32)
        if not last:
            h = jnp.maximum(h, 0.0)
        hb = h.astype(bf16)

    h1 = jnp.dot(hb, w1_ref[...], preferred_element_type=f32) + b1_ref[...]
    h1 = jnp.maximum(h1, 0.0).astype(bf16)
    y = jnp.dot(h1, w2_ref[...], preferred_element_type=f32) + b2_ref[...]
    o_ref[...] = y


def _blockdiag(wmat, k):
    return jnp.kron(jnp.eye(k, dtype=wmat.dtype), wmat)


@jax.jit
def kernel(x, dft_fwd, dft_inv, w0, b0, w1, b1, w2, b2,
           v2_0, wp_0, bp_0, v2_1, wp_1, bp_1, v2_2, wp_2, bp_2):
    bf16 = jnp.bfloat16
    f32 = jnp.float32
    B, s, cin0 = x.shape
    kb = _KB
    G = B // kb

    x4 = (x.reshape(G, kb, s, cin0).transpose(0, 2, 1, 3)
          .reshape(G, s, kb * cin0).astype(bf16))
    x4 = jnp.concatenate([x4, jnp.ones((G, s, 1), bf16)], axis=2)

    f_mat = dft_fwd.astype(bf16)
    if_aug = jnp.concatenate(
        [dft_inv, jnp.ones((s, 1), f32), jnp.zeros((s, 63), f32)],
        axis=1).astype(bf16)
    w0k = jnp.concatenate([_blockdiag(w0, kb), jnp.tile(b0, (1, kb))],
                          axis=0).astype(bf16)
    w1k = _blockdiag(w1, kb).astype(bf16)
    b1k = jnp.tile(b1, (1, kb))
    w2k = _blockdiag(w2, kb).astype(bf16)
    b2k = jnp.tile(b2, (1, kb))

    inputs = [x4, f_mat, if_aug, w0k]
    for v2, wp, bp in ((v2_0, wp_0, bp_0), (v2_1, wp_1, bp_1),
                       (v2_2, wp_2, bp_2)):
        wp_lo = _blockdiag(wp, kb)
        wp_hi = jnp.concatenate(
            [jnp.tile(bp, (1, kb)), jnp.zeros((63, kb * _W), f32)],
            axis=0)
        inputs += [v2.astype(bf16), wp_lo.astype(bf16), wp_hi.astype(bf16)]
    inputs += [w1k, b1k, w2k, b2k]

    def full(arr):
        shp = tuple(arr.shape)
        return pl.BlockSpec(shp, lambda b, _r=len(shp): (0,) * _r)

    in_specs = [pl.BlockSpec((pl.Squeezed(), s, kb * cin0 + 1),
                             lambda b: (b, 0, 0))]
    in_specs += [full(a) for a in inputs[1:]]

    out = pl.pallas_call(
        _body,
        out_shape=jax.ShapeDtypeStruct((G, s, kb), f32),
        grid=(G,),
        in_specs=in_specs,
        out_specs=pl.BlockSpec((pl.Squeezed(), s, kb), lambda b: (b, 0, 0)),
        compiler_params=pltpu.CompilerParams(
            dimension_semantics=("parallel",),
            vmem_limit_bytes=48 * 1024 * 1024,
        ),
    )(*inputs)

    return out.transpose(0, 2, 1).reshape(B, s, 1)
